# Initial kernel scaffold; baseline (speedup 1.0000x reference)
import jax
import jax.numpy as jnp
from jax.experimental import pallas as pl


def kernel(mem, idx, val):
    raise NotImplementedError("write your pallas kernel here")


# speedup vs baseline: 1.7353x; 1.7353x over previous
import functools

import jax
import jax.numpy as jnp
from jax import lax
from jax.experimental import pallas as pl
from jax.experimental.pallas import tpu as pltpu
from jax.experimental.pallas import tpu_sc as plsc

M = 1_000_000
D = 64
B = 16384
L = 16
NC, NS = 2, 16
NW = NC * NS
R = M // NW
NCHUNK = B // L
CW = 128
GROUPS = CW // L

_mesh = plsc.VectorSubcoreMesh(core_axis_name="c", subcore_axis_name="s")


@functools.partial(
    pl.kernel,
    mesh=_mesh,
    out_type=(),
    compiler_params=pltpu.CompilerParams(
        needs_layout_passes=False, use_tc_tiling_on_sc=False),
    scratch_types=[
        pltpu.VMEM((B,), jnp.int32),
        pltpu.VMEM((R,), jnp.int32),
        pltpu.VMEM((B,), jnp.int32),
        pltpu.VMEM((B,), jnp.int32),
        pltpu.VMEM((1, CW), jnp.int32),
        pltpu.VMEM((1, CW), jnp.int32),
        pltpu.VMEM((CW, D), jnp.float32),
        pltpu.SemaphoreType.DMA,
    ],
)
def _scatter(out_hbm, idx_hbm, val_hbm,
             idx_v, tab_v, sel_row, sel_pos, dma_row, dma_pos, rows_v, sem):
  c = lax.axis_index("c")
  s = lax.axis_index("s")
  wid = s * NC + c
  base = wid * R

  pltpu.sync_copy(idx_hbm, idx_v)

  lanes = lax.iota(jnp.int32, L)

  def mark(k, carry):
    iv = idx_v[pl.ds(k * L, L)]
    m = (iv >= base) & (iv < base + R)
    loc = jnp.clip(iv - base, 0, R - 1)
    pos = k * L + lanes
    plsc.store_scatter(tab_v, [loc], pos, mask=m)
    return carry

  lax.fori_loop(0, NCHUNK, mark, 0)

  def compact(k, cnt):
    iv = idx_v[pl.ds(k * L, L)]
    m = (iv >= base) & (iv < base + R)
    loc = jnp.clip(iv - base, 0, R - 1)
    pos = k * L + lanes
    g = plsc.load_gather(tab_v, [loc], mask=m)
    win = m & (g == pos)
    plsc.store_compressed(sel_row.at[pl.ds(cnt, L)], iv, mask=win)
    plsc.store_compressed(sel_pos.at[pl.ds(cnt, L)], pos, mask=win)
    return cnt + jnp.max(plsc.all_reduce_population_count(win))

  n = lax.fori_loop(0, NCHUNK, compact, jnp.int32(0))

  nch = (n + CW - 1) // CW

  def move(ci, carry):
    start = ci * CW
    last = n - 1
    for g in range(GROUPS):
      offs = jnp.minimum(start + g * L + lanes, last)
      dma_row[0, pl.ds(g * L, L)] = plsc.load_gather(sel_row, [offs])
      dma_pos[0, pl.ds(g * L, L)] = plsc.load_gather(sel_pos, [offs])
    pltpu.async_copy(val_hbm.at[dma_pos.at[0]], rows_v, sem).wait()
    pltpu.async_copy(rows_v, out_hbm.at[dma_row.at[0]], sem).wait()
    return carry

  lax.fori_loop(0, nch, move, 0)


def kernel(mem, idx, val):
  out = jax.new_ref(mem)
  _scatter(out, idx.astype(jnp.int32), val)
  return out[...]
